# Initial kernel scaffold; baseline (speedup 1.0000x reference)
import jax
import jax.numpy as jnp
from jax.experimental import pallas as pl


def kernel(tokens, token_embedding, position_embedding):
    raise NotImplementedError("write your pallas kernel here")


# speedup vs baseline: 3.9290x; 3.9290x over previous
import functools

import jax
import jax.numpy as jnp
from jax import lax
from jax.experimental import pallas as pl
from jax.experimental.pallas import tpu as pltpu
from jax.experimental.pallas import tpu_sc as plsc

B = 4096
T = 77
D = 128
LANES = 16
NC = 2
NS = 16
NW = NC * NS
SEQ_PER_W = B // NW
NBUF = 4


def _body(tok_hbm, table_hbm, pos_hbm, out_hbm,
          idx_v, pos_v, bufs,
          sg0, sg1, sg2, sg3, ss0, ss1, ss2, ss3):
  sem_g = (sg0, sg1, sg2, sg3)
  sem_s = (ss0, ss1, ss2, ss3)
  wid = lax.axis_index("s") * NC + lax.axis_index("c")
  seq0 = wid * SEQ_PER_W

  pltpu.sync_copy(tok_hbm.at[pl.ds(seq0, SEQ_PER_W)], idx_v)
  pltpu.sync_copy(pos_hbm, pos_v)

  def gather(s, b):
    return pltpu.make_async_copy(table_hbm.at[idx_v.at[s]], bufs.at[b],
                                 sem_g[b])

  def scatter(s, b):
    return pltpu.make_async_copy(bufs.at[b], out_hbm.at[seq0 + s], sem_s[b])

  def add_pos(b):
    buf = bufs.at[b]

    def row(r, carry):
      for c in range(D // LANES):
        sl = pl.ds(c * LANES, LANES)
        buf[r, sl] += pos_v[r, sl]
      return carry

    lax.fori_loop(0, T, row, 0, unroll=7)

  def step(s, b, refill, drain):
    b2 = (b + 2) % NBUF
    if refill:
      if drain:
        scatter(s - 2, b2).wait()
      gather(s + 2, b2).start()
    gather(s, b).wait()
    add_pos(b)
    scatter(s, b).start()

  gather(0, 0).start()
  gather(1, 1).start()

  for b in range(NBUF):
    step(b, b, refill=True, drain=(b >= 2))

  def outer(g, carry):
    for b in range(NBUF):
      step(g * NBUF + b, b, refill=True, drain=True)
    return carry

  lax.fori_loop(1, SEQ_PER_W // NBUF - 1, outer, 0)

  g = SEQ_PER_W // NBUF - 1
  for b in range(NBUF):
    step(g * NBUF + b, b, refill=(b < 2), drain=(b < 2))

  for b in range(NBUF):
    scatter(g * NBUF + b, b).wait()


_kern = functools.partial(
    pl.kernel,
    out_type=jax.ShapeDtypeStruct((B, T, D), jnp.float32),
    mesh=plsc.VectorSubcoreMesh(core_axis_name="c", subcore_axis_name="s"),
    scratch_types=[
        pltpu.VMEM((SEQ_PER_W, T), jnp.int32),
        pltpu.VMEM((T, D), jnp.float32),
        pltpu.VMEM((NBUF, T, D), jnp.float32),
    ] + [pltpu.SemaphoreType.DMA] * (2 * NBUF),
)(_body)


@jax.jit
def kernel(tokens, token_embedding, position_embedding):
  return _kern(tokens, token_embedding, position_embedding)


# speedup vs baseline: 3.9363x; 1.0019x over previous
import functools

import jax
import jax.numpy as jnp
from jax import lax
from jax.experimental import pallas as pl
from jax.experimental.pallas import tpu as pltpu
from jax.experimental.pallas import tpu_sc as plsc

B = 4096
T = 77
D = 128
LANES = 16
NC = 2
NS = 16
NW = NC * NS
SEQ_PER_W = B // NW
NBUF = 4


def _body(tok_hbm, table_hbm, pos_hbm, out_hbm,
          idx_v, pos_v, bufs,
          sg0, sg1, sg2, sg3, ss0, ss1, ss2, ss3):
  sem_g = (sg0, sg1, sg2, sg3)
  sem_s = (ss0, ss1, ss2, ss3)
  wid = lax.axis_index("s") * NC + lax.axis_index("c")
  seq0 = wid * SEQ_PER_W

  pltpu.sync_copy(tok_hbm.at[pl.ds(seq0, SEQ_PER_W)], idx_v)
  pltpu.sync_copy(pos_hbm, pos_v)

  def gather(s, b):
    return pltpu.make_async_copy(table_hbm.at[idx_v.at[s, pl.ds(0, T)]],
                                 bufs.at[b], sem_g[b])

  def scatter(s, b):
    return pltpu.make_async_copy(bufs.at[b], out_hbm.at[seq0 + s], sem_s[b])

  def add_pos(b):
    buf = bufs.at[b]

    def row(r, carry):
      for c in range(D // LANES):
        sl = pl.ds(c * LANES, LANES)
        buf[r, sl] += pos_v[r, sl]
      return carry

    lax.fori_loop(0, T, row, 0, unroll=7)

  def step(s, b, refill, drain):
    b2 = (b + 2) % NBUF
    if refill:
      if drain:
        scatter(s - 2, b2).wait()
      gather(s + 2, b2).start()
    gather(s, b).wait()
    add_pos(b)
    scatter(s, b).start()

  gather(0, 0).start()
  gather(1, 1).start()

  for b in range(NBUF):
    step(b, b, refill=True, drain=(b >= 2))

  def outer(g, carry):
    for b in range(NBUF):
      step(g * NBUF + b, b, refill=True, drain=True)
    return carry

  lax.fori_loop(1, SEQ_PER_W // NBUF - 1, outer, 0)

  g = SEQ_PER_W // NBUF - 1
  for b in range(NBUF):
    step(g * NBUF + b, b, refill=(b < 2), drain=(b < 2))

  for b in range(NBUF):
    scatter(g * NBUF + b, b).wait()


_kern = functools.partial(
    pl.kernel,
    out_type=jax.ShapeDtypeStruct((B, T, D), jnp.float32),
    mesh=plsc.VectorSubcoreMesh(core_axis_name="c", subcore_axis_name="s"),
    compiler_params=pltpu.CompilerParams(use_tc_tiling_on_sc=True),
    scratch_types=[
        pltpu.VMEM((SEQ_PER_W, D), jnp.int32),
        pltpu.VMEM((T, D), jnp.float32),
        pltpu.VMEM((NBUF, T, D), jnp.float32),
    ] + [pltpu.SemaphoreType.DMA] * (2 * NBUF),
)(_body)


@jax.jit
def kernel(tokens, token_embedding, position_embedding):
  tokens_padded = jnp.pad(tokens, ((0, 0), (0, D - T)))
  return _kern(tokens_padded, token_embedding, position_embedding)


# speedup vs baseline: 4.0723x; 1.0345x over previous
import functools

import jax
import jax.numpy as jnp
from jax import lax
from jax.experimental import pallas as pl
from jax.experimental.pallas import tpu as pltpu
from jax.experimental.pallas import tpu_sc as plsc

B = 4096
T = 77
D = 128
NC = 2
NS = 16
NW = NC * NS
SEQ_PER_W = B // NW
NBUF = 4


def _body(tok_hbm, table_hbm, out_hbm,
          idx_v, bufs,
          sg0, sg1, sg2, sg3, ss0, ss1, ss2, ss3):
  sem_g = (sg0, sg1, sg2, sg3)
  sem_s = (ss0, ss1, ss2, ss3)
  wid = lax.axis_index("s") * NC + lax.axis_index("c")
  seq0 = wid * SEQ_PER_W

  pltpu.sync_copy(tok_hbm.at[pl.ds(seq0, SEQ_PER_W)], idx_v)

  def gather(s, b):
    return pltpu.make_async_copy(table_hbm.at[idx_v.at[s]], bufs.at[b],
                                 sem_g[b])

  def scatter(s, b):
    return pltpu.make_async_copy(bufs.at[b], out_hbm.at[seq0 + s], sem_s[b])

  def step(s, b, refill, drain):
    b2 = (b + 2) % NBUF
    if refill:
      if drain:
        scatter(s - 2, b2).wait()
      gather(s + 2, b2).start()
    gather(s, b).wait()
    scatter(s, b).start()

  gather(0, 0).start()
  gather(1, 1).start()

  for b in range(NBUF):
    step(b, b, refill=True, drain=(b >= 2))

  def outer(g, carry):
    for b in range(NBUF):
      step(g * NBUF + b, b, refill=True, drain=True)
    return carry

  lax.fori_loop(1, SEQ_PER_W // NBUF - 1, outer, 0)

  g = SEQ_PER_W // NBUF - 1
  for b in range(NBUF):
    step(g * NBUF + b, b, refill=(b < 2), drain=(b < 2))

  for b in range(NBUF):
    scatter(g * NBUF + b, b).wait()


_kern = functools.partial(
    pl.kernel,
    out_type=jax.ShapeDtypeStruct((B, T, D), jnp.float32),
    mesh=plsc.VectorSubcoreMesh(core_axis_name="c", subcore_axis_name="s"),
    scratch_types=[
        pltpu.VMEM((SEQ_PER_W, T), jnp.int32),
        pltpu.VMEM((NBUF, T, D), jnp.float32),
    ] + [pltpu.SemaphoreType.DMA] * (2 * NBUF),
)(_body)


@jax.jit
def kernel(tokens, token_embedding, position_embedding):
  gathered = _kern(tokens, token_embedding)
  return gathered + position_embedding[None, :, :]
